# baseline (speedup 1.0000x reference)
import jax
import jax.numpy as jnp
from jax import lax
from jax.experimental import pallas as pl
from jax.experimental.pallas import tpu as pltpu
from jax.experimental.pallas import tpu_sc as plsc

_B = 16384
_K = 64
_NC = 2
_NS = 16
_NW = _NC * _NS
_BPW = _B // _NW
_CH = 128
_NCH = _BPW // _CH
_L = 16
_G = 16
_NG = _BPW // _G
_KP = 2 * _K


def _bias_body(uid_h, iid_h, ub_h, ib_h, out_h, uidx, iidx, ubg, ibg, outv, sem):
    wid = lax.axis_index("s") * _NC + lax.axis_index("c")
    base = wid * _BPW
    for c in range(_NCH):
        pltpu.sync_copy(uid_h.at[pl.ds(base + c * _CH, _CH)], uidx.at[c])
        pltpu.sync_copy(iid_h.at[pl.ds(base + c * _CH, _CH)], iidx.at[c])
    cps = []
    for c in range(_NCH):
        cps.append(pltpu.async_copy(ub_h.at[uidx.at[c]], ubg.at[pl.ds(c * _CH, _CH)], sem))
        cps.append(pltpu.async_copy(ib_h.at[iidx.at[c]], ibg.at[pl.ds(c * _CH, _CH)], sem))
    for cp in cps:
        cp.wait()

    def body(i, carry):
        outv[pl.ds(i * _L, _L)] = ubg[pl.ds(i * _L, _L)] + ibg[pl.ds(i * _L, _L)]
        return carry

    lax.fori_loop(0, _BPW // _L, body, 0)
    pltpu.sync_copy(outv, out_h.at[pl.ds(base, _BPW)])


def _dot_body(uid_h, iid_h, uf_h, if_h, bs_h, out_h,
              uidx, iidx, bsv, ru0, ru1, ri0, ri1, outv, semu, semi):
    wid = lax.axis_index("s") * _NC + lax.axis_index("c")
    base = wid * _BPW
    pltpu.sync_copy(uid_h.at[pl.ds(base, _BPW)], uidx)
    pltpu.sync_copy(iid_h.at[pl.ds(base, _BPW)], iidx)
    pltpu.sync_copy(bs_h.at[pl.ds(base, _BPW)], bsv)

    def fire(g, ru, ri):
        r0 = g * _G
        uvec = uidx[pl.ds(r0, _G)]
        ivec = iidx[pl.ds(r0, _G)]
        for l in range(_G):
            pltpu.async_copy(uf_h.at[lax.shift_right_logical(uvec[l], 1)], ru.at[l], semu)
            pltpu.async_copy(if_h.at[lax.shift_right_logical(ivec[l], 1)], ri.at[l], semi)

    def drain(ru, ri):
        pltpu.make_async_copy(uf_h.at[pl.ds(0, _G)], ru, semu).wait()
        pltpu.make_async_copy(if_h.at[pl.ds(0, _G)], ri, semi).wait()

    lanes = lax.iota(jnp.int32, _L)

    def compute(g, ru, ri):
        r0 = g * _G
        upar = lax.bitwise_and(uidx[pl.ds(r0, _G)], 1) * _K
        ipar = lax.bitwise_and(iidx[pl.ds(r0, _G)], 1) * _K
        acc = bsv[pl.ds(r0, _G)]
        for j in range(_K):
            acc = acc + (plsc.load_gather(ru, [lanes, upar + j])
                         * plsc.load_gather(ri, [lanes, ipar + j]))
        outv[pl.ds(r0, _G)] = acc

    fire(0, ru0, ri0)

    def pair(h, carry):
        g0 = 2 * h
        g1 = g0 + 1

        @pl.when(g1 < _NG)
        def _():
            fire(g1, ru1, ri1)

        drain(ru0, ri0)
        compute(g0, ru0, ri0)

        @pl.when(g0 + 2 < _NG)
        def _():
            fire(g0 + 2, ru0, ri0)

        @pl.when(g1 < _NG)
        def _():
            drain(ru1, ri1)
            compute(g1, ru1, ri1)

        return carry

    lax.fori_loop(0, (_NG + 1) // 2, pair, 0)
    pltpu.sync_copy(outv, out_h.at[pl.ds(base, _BPW)])


def kernel(user_id, item_id, user_factors, item_factors, user_bias, item_bias):
    uid = user_id.reshape(_B)
    iid = item_id.reshape(_B)
    ufp = user_factors.reshape(user_factors.shape[0] // 2, _KP)
    ifp = item_factors.reshape(item_factors.shape[0] // 2, _KP)
    mesh = plsc.VectorSubcoreMesh(core_axis_name="c", subcore_axis_name="s")

    bias_call = pl.kernel(
        _bias_body,
        out_type=jax.ShapeDtypeStruct((_B,), jnp.float32),
        mesh=mesh,
        scratch_types=[
            pltpu.VMEM((_NCH, _CH), jnp.int32),
            pltpu.VMEM((_NCH, _CH), jnp.int32),
            pltpu.VMEM((_BPW,), jnp.float32),
            pltpu.VMEM((_BPW,), jnp.float32),
            pltpu.VMEM((_BPW,), jnp.float32),
            pltpu.SemaphoreType.DMA,
        ],
        compiler_params=pltpu.CompilerParams(
            needs_layout_passes=False, use_tc_tiling_on_sc=False),
    )
    bias_sum = bias_call(uid, iid, user_bias.reshape(-1), item_bias.reshape(-1))

    dot_call = pl.kernel(
        _dot_body,
        out_type=jax.ShapeDtypeStruct((_B,), jnp.float32),
        mesh=mesh,
        scratch_types=[
            pltpu.VMEM((_BPW,), jnp.int32),
            pltpu.VMEM((_BPW,), jnp.int32),
            pltpu.VMEM((_BPW,), jnp.float32),
            pltpu.VMEM((_G, _KP), jnp.float32),
            pltpu.VMEM((_G, _KP), jnp.float32),
            pltpu.VMEM((_G, _KP), jnp.float32),
            pltpu.VMEM((_G, _KP), jnp.float32),
            pltpu.VMEM((_BPW,), jnp.float32),
            pltpu.SemaphoreType.DMA,
            pltpu.SemaphoreType.DMA,
        ],
        compiler_params=pltpu.CompilerParams(
            needs_layout_passes=False, use_tc_tiling_on_sc=True),
    )
    return dot_call(uid, iid, ufp, ifp, bias_sum)
